# baseline (speedup 1.0000x reference)
import functools

import jax
import jax.numpy as jnp
from jax import lax
from jax.experimental import pallas as pl
from jax.experimental.pallas import tpu as pltpu
from jax.experimental.pallas import tpu_sc as plsc

B, S, D, V = 1024, 200, 64, 100000
NC, NS = 2, 16
LANE = 128
NBR = B // LANE
NBPC = D // 8 // NC
SCH = 40
NSCH = S // SCH
NT = SCH // 2

_mesh = plsc.VectorSubcoreMesh(core_axis_name="c", subcore_axis_name="s")


@functools.partial(
    pl.kernel,
    mesh=_mesh,
    out_type=jax.ShapeDtypeStruct((S, D // 8, NBR, 8, LANE), jnp.float32),
    compiler_params=pltpu.CompilerParams(use_tc_tiling_on_sc=False,
                                         needs_layout_passes=False),
    scratch_types=(
        [pltpu.VMEM_SHARED((2, V, 8), jnp.float32),
         pltpu.VMEM((SCH, LANE), jnp.int32),
         pltpu.VMEM((8, 512), jnp.float32)]
        + [pltpu.VMEM((LANE, 8), jnp.float32) for _ in range(2)]
        + [pltpu.VMEM((8, LANE), jnp.float32) for _ in range(2)]
        + [pltpu.SemaphoreType.DMA for _ in range(5)]
    ),
)
def _embedT(ids_hbm, table_hbm, pe_hbm, out_hbm, spm, ids_v, pe_v, *rest):
    buf2 = list(rest[0:2])
    blk = list(rest[2:4])
    gsem = list(rest[4:6])
    osem = list(rest[6:8])
    ssem = rest[8]
    core = lax.axis_index("c")
    tid = lax.axis_index("s")
    bl = tid // NBR
    bt = tid % NBR

    def run_round(r):
        band = core * NBPC + 2 * r + bl
        @pl.when(bt == 0)
        def _():
            pltpu.async_copy(table_hbm.at[:, pl.ds(band * 8, 8)],
                             spm.at[bl], ssem).wait()
        pltpu.sync_copy(pe_hbm.at[pl.ds(band * 8, 8)], pe_v)
        plsc.subcore_barrier()

        def fire_gather(i, j):
            pltpu.async_copy(spm.at[bl].at[ids_v.at[i]], buf2[j], gsem[j])

        def wait_gather(i, j):
            pltpu.make_async_copy(spm.at[bl].at[ids_v.at[i]], buf2[j],
                                  gsem[j]).wait()

        def fire_out(sg, j):
            pltpu.async_copy(blk[j], out_hbm.at[sg, band, bt], osem[j])

        def wait_out(j):
            pltpu.make_async_copy(blk[j], out_hbm.at[0, 0, 0],
                                  osem[j]).wait()

        def transpose_add(sg, j):
            colv = jnp.full((16,), sg, jnp.int32)
            for dloc in range(8):
                dv = jnp.full((16,), dloc, jnp.int32)
                bvec = plsc.load_gather(pe_v, [dv, colv])
                for k in range(LANE // 16):
                    ridx = lax.iota(jnp.int32, 16) + (k * 16)
                    v = plsc.load_gather(buf2[j], [ridx, dv])
                    blk[j][dloc, pl.ds(k * 16, 16)] = v + bvec

        def do_pos(s0, i, j, first, last):
            if not last:
                fire_gather(i + 1, 1 - j)
            wait_gather(i, j)
            if not first:
                wait_out(j)
            transpose_add(s0 + i, j)
            fire_out(s0 + i, j)

        @pl.loop(0, NSCH)
        def _(sc):
            s0 = sc * SCH
            pltpu.sync_copy(
                ids_hbm.at[pl.ds(s0, SCH), pl.ds(bt * LANE, LANE)], ids_v)
            fire_gather(0, 0)
            do_pos(s0, 0, 0, first=True, last=False)
            do_pos(s0, 1, 1, first=True, last=False)

            @pl.loop(1, NT - 1)
            def _(t):
                do_pos(s0, 2 * t, 0, first=False, last=False)
                do_pos(s0, 2 * t + 1, 1, first=False, last=False)

            do_pos(s0, SCH - 2, 0, first=False, last=False)
            do_pos(s0, SCH - 1, 1, first=False, last=True)
            wait_out(0)
            wait_out(1)

        plsc.subcore_barrier()

    run_round(0)
    run_round(1)


def kernel(input_ids, token_table, pe):
    out5 = _embedT(input_ids.T, token_table, pe.T)
    return jnp.transpose(out5, (2, 4, 0, 1, 3)).reshape(B, S, D)


# speedup vs baseline: 1.9148x; 1.9148x over previous
import functools

import jax
import jax.numpy as jnp
from jax import lax
from jax.experimental import pallas as pl
from jax.experimental.pallas import tpu as pltpu
from jax.experimental.pallas import tpu_sc as plsc

B, S, D, V = 1024, 200, 64, 100000
NC, NS = 2, 16
LANE = 128
NBR = B // LANE
NBPC = D // 8 // NC
SCH = 40
NSCH = S // SCH
NT = SCH // 2

_mesh = plsc.VectorSubcoreMesh(core_axis_name="c", subcore_axis_name="s")


@functools.partial(
    pl.kernel,
    mesh=_mesh,
    out_type=jax.ShapeDtypeStruct((S, D // 8, NBR, 8, LANE), jnp.float32),
    compiler_params=pltpu.CompilerParams(use_tc_tiling_on_sc=False,
                                         needs_layout_passes=False),
    scratch_types=(
        [pltpu.VMEM_SHARED((2, 8 * V), jnp.float32),
         pltpu.VMEM((SCH, LANE), jnp.int32),
         pltpu.VMEM((8, 512), jnp.float32)]
        + [pltpu.VMEM((8, LANE), jnp.float32) for _ in range(2)]
        + [pltpu.SemaphoreType.DMA for _ in range(5)]
    ),
)
def _embedT(ids_hbm, table_hbm, pe_hbm, out_hbm, spm, ids_v, pe_v, *rest):
    blk = list(rest[0:2])
    gsem = list(rest[2:4])
    osem = list(rest[4:6])
    ssem = rest[6]
    core = lax.axis_index("c")
    tid = lax.axis_index("s")
    bl = tid // NBR
    bt = tid % NBR

    def run_round(r):
        band = core * NBPC + 2 * r + bl
        @pl.when(bt == 0)
        def _():
            pltpu.async_copy(table_hbm.at[band], spm.at[bl], ssem).wait()
        pltpu.sync_copy(pe_hbm.at[pl.ds(band * 8, 8)], pe_v)
        plsc.subcore_barrier()

        def fire_gather(i, jj):
            for dloc in range(8):
                pltpu.async_copy(
                    spm.at[bl, pl.ds(dloc * V, V)].at[ids_v.at[i]],
                    blk[jj].at[dloc], gsem[jj])

        def wait_gather(i, jj):
            for dloc in range(8):
                pltpu.make_async_copy(
                    spm.at[bl, pl.ds(dloc * V, V)].at[ids_v.at[i]],
                    blk[jj].at[dloc], gsem[jj]).wait()

        def fire_out(sg, j):
            pltpu.async_copy(blk[j], out_hbm.at[sg, band, bt], osem[j])

        def wait_out(j):
            pltpu.make_async_copy(blk[j], out_hbm.at[0, 0, 0],
                                  osem[j]).wait()

        def add_pe(sg, j):
            colv = jnp.full((16,), sg, jnp.int32)
            for dloc in range(8):
                dv = jnp.full((16,), dloc, jnp.int32)
                bvec = plsc.load_gather(pe_v, [dv, colv])
                for k in range(LANE // 16):
                    plsc.addupdate(
                        blk[j].at[dloc, pl.ds(k * 16, 16)], bvec)

        def do_pos(s0, i, j, first, last):
            if not last:
                if not first:
                    wait_out(1 - j)
                fire_gather(i + 1, 1 - j)
            wait_gather(i, j)
            add_pe(s0 + i, j)
            fire_out(s0 + i, j)

        @pl.loop(0, NSCH)
        def _(sc):
            s0 = sc * SCH
            pltpu.sync_copy(
                ids_hbm.at[pl.ds(s0, SCH), pl.ds(bt * LANE, LANE)], ids_v)
            fire_gather(0, 0)
            do_pos(s0, 0, 0, first=True, last=False)
            do_pos(s0, 1, 1, first=False, last=False)

            @pl.loop(1, NT - 1)
            def _(t):
                do_pos(s0, 2 * t, 0, first=False, last=False)
                do_pos(s0, 2 * t + 1, 1, first=False, last=False)

            do_pos(s0, SCH - 2, 0, first=False, last=False)
            do_pos(s0, SCH - 1, 1, first=False, last=True)
            wait_out(0)
            wait_out(1)

        plsc.subcore_barrier()

    run_round(0)
    run_round(1)


def kernel(input_ids, token_table, pe):
    tb = token_table.reshape(V, 8, 8).transpose(1, 2, 0).reshape(8, 8 * V)
    out5 = _embedT(input_ids.T, tb, pe.T)
    return jnp.transpose(out5, (2, 4, 0, 1, 3)).reshape(B, S, D)


# speedup vs baseline: 1.9475x; 1.0171x over previous
import functools

import jax
import jax.numpy as jnp
from jax import lax
from jax.experimental import pallas as pl
from jax.experimental.pallas import tpu as pltpu
from jax.experimental.pallas import tpu_sc as plsc

B, S, D, V = 1024, 200, 64, 100000
NC, NS = 2, 16
LANE = 128
NBR = B // LANE
NBPC = D // 8 // NC
SCH = 40
NSCH = S // SCH
NT = SCH // 2

_mesh = plsc.VectorSubcoreMesh(core_axis_name="c", subcore_axis_name="s")


@functools.partial(
    pl.kernel,
    mesh=_mesh,
    out_type=jax.ShapeDtypeStruct((S, D // 8, NBR, 8, LANE), jnp.float32),
    compiler_params=pltpu.CompilerParams(use_tc_tiling_on_sc=False,
                                         needs_layout_passes=False),
    scratch_types=(
        [pltpu.VMEM_SHARED((2, 8 * V), jnp.float32),
         pltpu.VMEM((SCH * LANE,), jnp.int32),
         pltpu.VMEM((8, 512), jnp.float32)]
        + [pltpu.VMEM((8, 2 * LANE), jnp.float32) for _ in range(2)]
        + [pltpu.SemaphoreType.DMA for _ in range(5)]
    ),
)
def _embedT(ids_hbm, table_hbm, pe_hbm, out_hbm, spm, ids_v, pe_v, *rest):
    blk = list(rest[0:2])
    gsem = list(rest[2:4])
    osem = list(rest[4:6])
    ssem = rest[6]
    core = lax.axis_index("c")
    tid = lax.axis_index("s")
    bl = tid // NBR
    bt = tid % NBR

    def run_round(r):
        band = core * NBPC + 2 * r + bl
        @pl.when(bt == 0)
        def _():
            pltpu.async_copy(table_hbm.at[band], spm.at[bl], ssem).wait()
        pltpu.sync_copy(pe_hbm.at[pl.ds(band * 8, 8)], pe_v)
        plsc.subcore_barrier()

        def fire_gather(p, jj):
            idx = ids_v.at[pl.ds(p * 2 * LANE, 2 * LANE)]
            for dloc in range(8):
                pltpu.async_copy(
                    spm.at[bl, pl.ds(dloc * V, V)].at[idx],
                    blk[jj].at[dloc], gsem[jj])

        def wait_gather(p, jj):
            idx = ids_v.at[pl.ds(p * 2 * LANE, 2 * LANE)]
            for dloc in range(8):
                pltpu.make_async_copy(
                    spm.at[bl, pl.ds(dloc * V, V)].at[idx],
                    blk[jj].at[dloc], gsem[jj]).wait()

        def fire_out(sg, j):
            for h in range(2):
                pltpu.async_copy(blk[j].at[:, pl.ds(h * LANE, LANE)],
                                 out_hbm.at[sg + h, band, bt], osem[j])

        def wait_out(j):
            for _ in range(2):
                pltpu.make_async_copy(blk[j].at[:, pl.ds(0, LANE)],
                                      out_hbm.at[0, 0, 0], osem[j]).wait()

        def add_pe(sg, j):
            for h in range(2):
                colv = jnp.full((16,), sg + h, jnp.int32)
                for dloc in range(8):
                    dv = jnp.full((16,), dloc, jnp.int32)
                    bvec = plsc.load_gather(pe_v, [dv, colv])
                    for k in range(LANE // 16):
                        plsc.addupdate(
                            blk[j].at[dloc, pl.ds(h * LANE + k * 16, 16)],
                            bvec)

        def do_pair(s0, p, j, first, last):
            if not last:
                if not first:
                    wait_out(1 - j)
                fire_gather(p + 1, 1 - j)
            wait_gather(p, j)
            add_pe(s0 + 2 * p, j)
            fire_out(s0 + 2 * p, j)

        @pl.loop(0, NSCH)
        def _(sc):
            s0 = sc * SCH
            pltpu.sync_copy(
                ids_hbm.at[bt, pl.ds(s0 * LANE, SCH * LANE)], ids_v)
            fire_gather(0, 0)
            do_pair(s0, 0, 0, first=True, last=False)
            do_pair(s0, 1, 1, first=False, last=False)

            @pl.loop(1, NT // 2 - 1)
            def _(t):
                do_pair(s0, 2 * t, 0, first=False, last=False)
                do_pair(s0, 2 * t + 1, 1, first=False, last=False)

            do_pair(s0, NT - 2, 0, first=False, last=False)
            do_pair(s0, NT - 1, 1, first=False, last=True)
            wait_out(0)
            wait_out(1)

        plsc.subcore_barrier()

    run_round(0)
    run_round(1)


def kernel(input_ids, token_table, pe):
    tb = token_table.reshape(V, 8, 8).transpose(1, 2, 0).reshape(8, 8 * V)
    idsp = (input_ids.T.reshape(S, NBR, LANE).transpose(1, 0, 2)
            .reshape(NBR, S * LANE))
    out5 = _embedT(idsp, tb, pe.T)
    return jnp.transpose(out5, (2, 4, 0, 1, 3)).reshape(B, S, D)


# speedup vs baseline: 2.0241x; 1.0393x over previous
import functools

import jax
import jax.numpy as jnp
from jax import lax
from jax.experimental import pallas as pl
from jax.experimental.pallas import tpu as pltpu
from jax.experimental.pallas import tpu_sc as plsc

B, S, D, V = 1024, 200, 64, 100000
NC, NS = 2, 16
LANE = 128
NBR = B // LANE
NBPC = D // 8 // NC
SCH = 40
NSCH = S // SCH
NT = SCH // 2

_mesh = plsc.VectorSubcoreMesh(core_axis_name="c", subcore_axis_name="s")


@functools.partial(
    pl.kernel,
    mesh=_mesh,
    out_type=jax.ShapeDtypeStruct((S, D // 8, NBR, 8, LANE), jnp.float32),
    compiler_params=pltpu.CompilerParams(use_tc_tiling_on_sc=False,
                                         needs_layout_passes=False),
    scratch_types=(
        [pltpu.VMEM_SHARED((2, 8 * V), jnp.float32),
         pltpu.VMEM((SCH * LANE,), jnp.int32),
         pltpu.VMEM((8, 512), jnp.float32)]
        + [pltpu.VMEM((8, 2 * LANE), jnp.float32) for _ in range(2)]
        + [pltpu.SemaphoreType.DMA for _ in range(9)]
    ),
)
def _embedT(ids_hbm, table_hbm, pe_hbm, out_hbm, spm, ids_v, pe_v, *rest):
    blk = list(rest[0:2])
    gsem = list(rest[2:6])
    osem = list(rest[6:8])
    ssem = rest[8]
    core = lax.axis_index("c")
    tid = lax.axis_index("s")
    bl = tid // NBR
    bt = tid % NBR

    def run_round(r):
        band = core * NBPC + 2 * r + bl
        @pl.when(bt == 0)
        def _():
            pltpu.async_copy(table_hbm.at[band], spm.at[bl], ssem).wait()
        pltpu.sync_copy(pe_hbm.at[pl.ds(band * 8, 8)], pe_v)
        plsc.subcore_barrier()

        def fire_gather(p, jj):
            idx = ids_v.at[pl.ds(p * 2 * LANE, 2 * LANE)]
            for dloc in range(8):
                pltpu.async_copy(
                    spm.at[bl, pl.ds(dloc * V, V)].at[idx],
                    blk[jj].at[dloc], gsem[2 * jj + dloc % 2])

        def wait_gather(p, jj):
            idx = ids_v.at[pl.ds(p * 2 * LANE, 2 * LANE)]
            for dloc in range(8):
                pltpu.make_async_copy(
                    spm.at[bl, pl.ds(dloc * V, V)].at[idx],
                    blk[jj].at[dloc], gsem[2 * jj + dloc % 2]).wait()

        def fire_out(sg, j):
            for h in range(2):
                pltpu.async_copy(blk[j].at[:, pl.ds(h * LANE, LANE)],
                                 out_hbm.at[sg + h, band, bt], osem[j])

        def wait_out(j):
            for _ in range(2):
                pltpu.make_async_copy(blk[j].at[:, pl.ds(0, LANE)],
                                      out_hbm.at[0, 0, 0], osem[j]).wait()

        def add_pe(sg, j):
            for h in range(2):
                colv = jnp.full((16,), sg + h, jnp.int32)
                for dloc in range(8):
                    dv = jnp.full((16,), dloc, jnp.int32)
                    bvec = plsc.load_gather(pe_v, [dv, colv])
                    for k in range(LANE // 16):
                        plsc.addupdate(
                            blk[j].at[dloc, pl.ds(h * LANE + k * 16, 16)],
                            bvec)

        def do_pair(s0, p, j, first, last):
            if not last:
                if not first:
                    wait_out(1 - j)
                fire_gather(p + 1, 1 - j)
            wait_gather(p, j)
            add_pe(s0 + 2 * p, j)
            fire_out(s0 + 2 * p, j)

        @pl.loop(0, NSCH)
        def _(sc):
            s0 = sc * SCH
            pltpu.sync_copy(
                ids_hbm.at[bt, pl.ds(s0 * LANE, SCH * LANE)], ids_v)
            fire_gather(0, 0)
            do_pair(s0, 0, 0, first=True, last=False)
            do_pair(s0, 1, 1, first=False, last=False)

            @pl.loop(1, NT // 2 - 1)
            def _(t):
                do_pair(s0, 2 * t, 0, first=False, last=False)
                do_pair(s0, 2 * t + 1, 1, first=False, last=False)

            do_pair(s0, NT - 2, 0, first=False, last=False)
            do_pair(s0, NT - 1, 1, first=False, last=True)
            wait_out(0)
            wait_out(1)

        plsc.subcore_barrier()

    run_round(0)
    run_round(1)


def kernel(input_ids, token_table, pe):
    tb = token_table.reshape(V, 8, 8).transpose(1, 2, 0).reshape(8, 8 * V)
    idsp = (input_ids.T.reshape(S, NBR, LANE).transpose(1, 0, 2)
            .reshape(NBR, S * LANE))
    out5 = _embedT(idsp, tb, pe.T)
    return jnp.transpose(out5, (2, 4, 0, 1, 3)).reshape(B, S, D)


# speedup vs baseline: 2.0882x; 1.0317x over previous
import functools

import jax
import jax.numpy as jnp
from jax import lax
from jax.experimental import pallas as pl
from jax.experimental.pallas import tpu as pltpu
from jax.experimental.pallas import tpu_sc as plsc

B, S, D, V = 1024, 200, 64, 100000
NC, NS = 2, 16
LANE = 128
NBR = B // LANE
NBPC = D // 8 // NC
SCH = 100
NSCH = S // SCH
NT = SCH // 2

_mesh = plsc.VectorSubcoreMesh(core_axis_name="c", subcore_axis_name="s")


@functools.partial(
    pl.kernel,
    mesh=_mesh,
    out_type=jax.ShapeDtypeStruct((S, D // 8, NBR, 8, LANE), jnp.float32),
    compiler_params=pltpu.CompilerParams(use_tc_tiling_on_sc=False,
                                         needs_layout_passes=False),
    scratch_types=(
        [pltpu.VMEM_SHARED((2, 8 * V), jnp.float32),
         pltpu.VMEM((SCH * LANE,), jnp.int32),
         pltpu.VMEM((8, 512), jnp.float32)]
        + [pltpu.VMEM((8, 2 * LANE), jnp.float32) for _ in range(2)]
        + [pltpu.SemaphoreType.DMA for _ in range(9)]
    ),
)
def _embedT(ids_hbm, table_hbm, pe_hbm, out_hbm, spm, ids_v, pe_v, *rest):
    blk = list(rest[0:2])
    gsem = list(rest[2:6])
    osem = list(rest[6:8])
    ssem = rest[8]
    core = lax.axis_index("c")
    tid = lax.axis_index("s")
    bl = tid // NBR
    bt = tid % NBR

    def run_round(r):
        band = core * NBPC + 2 * r + bl
        @pl.when(bt == 0)
        def _():
            pltpu.async_copy(table_hbm.at[band], spm.at[bl], ssem).wait()
        pltpu.sync_copy(pe_hbm.at[pl.ds(band * 8, 8)], pe_v)
        plsc.subcore_barrier()

        def fire_gather(p, jj):
            idx = ids_v.at[pl.ds(p * 2 * LANE, 2 * LANE)]
            for dloc in range(8):
                pltpu.async_copy(
                    spm.at[bl, pl.ds(dloc * V, V)].at[idx],
                    blk[jj].at[dloc], gsem[2 * jj + dloc % 2])

        def wait_gather(p, jj):
            idx = ids_v.at[pl.ds(p * 2 * LANE, 2 * LANE)]
            for dloc in range(8):
                pltpu.make_async_copy(
                    spm.at[bl, pl.ds(dloc * V, V)].at[idx],
                    blk[jj].at[dloc], gsem[2 * jj + dloc % 2]).wait()

        def fire_out(sg, j):
            for h in range(2):
                pltpu.async_copy(blk[j].at[:, pl.ds(h * LANE, LANE)],
                                 out_hbm.at[sg + h, band, bt], osem[j])

        def wait_out(j):
            for _ in range(2):
                pltpu.make_async_copy(blk[j].at[:, pl.ds(0, LANE)],
                                      out_hbm.at[0, 0, 0], osem[j]).wait()

        def add_pe(sg, j):
            for h in range(2):
                colv = jnp.full((16,), sg + h, jnp.int32)
                for dloc in range(8):
                    dv = jnp.full((16,), dloc, jnp.int32)
                    bvec = plsc.load_gather(pe_v, [dv, colv])
                    for k in range(LANE // 16):
                        plsc.addupdate(
                            blk[j].at[dloc, pl.ds(h * LANE + k * 16, 16)],
                            bvec)

        def do_pair(s0, p, j, first, last):
            if not last:
                if not first:
                    wait_out(1 - j)
                fire_gather(p + 1, 1 - j)
            wait_gather(p, j)
            add_pe(s0 + 2 * p, j)
            fire_out(s0 + 2 * p, j)

        @pl.loop(0, NSCH)
        def _(sc):
            s0 = sc * SCH
            pltpu.sync_copy(
                ids_hbm.at[bt, pl.ds(s0 * LANE, SCH * LANE)], ids_v)
            fire_gather(0, 0)
            do_pair(s0, 0, 0, first=True, last=False)
            do_pair(s0, 1, 1, first=False, last=False)

            @pl.loop(1, NT // 2 - 1)
            def _(t):
                do_pair(s0, 2 * t, 0, first=False, last=False)
                do_pair(s0, 2 * t + 1, 1, first=False, last=False)

            do_pair(s0, NT - 2, 0, first=False, last=False)
            do_pair(s0, NT - 1, 1, first=False, last=True)
            wait_out(0)
            wait_out(1)

        plsc.subcore_barrier()

    run_round(0)
    run_round(1)


def kernel(input_ids, token_table, pe):
    tb = token_table.reshape(V, 8, 8).transpose(1, 2, 0).reshape(8, 8 * V)
    idsp = (input_ids.T.reshape(S, NBR, LANE).transpose(1, 0, 2)
            .reshape(NBR, S * LANE))
    out5 = _embedT(idsp, tb, pe.T)
    return jnp.transpose(out5, (2, 4, 0, 1, 3)).reshape(B, S, D)
